# baseline (speedup 1.0000x reference)
import functools

import jax
import jax.numpy as jnp
from jax import lax
from jax.experimental import pallas as pl
from jax.experimental.pallas import tpu as pltpu
from jax.experimental.pallas import tpu_sc as plsc

N = 10000
E = 320000
H = 128
NC = 2
NS = 16
K = 128
NB1 = 3
CW1 = 159
E_PAD1 = CW1 * NS * K
NB2 = 2
CW2 = 80
E_PAD2 = CW2 * NS * NC * K
HALF = N // NC
ACC_ROWS = 5120
DUMP = HALF + 8
RPT = ACC_ROWS // NS
DB = 16
N_H = 10128
N_ACC2 = 10112
RPT2 = N_ACC2 // NS

_mesh = plsc.VectorSubcoreMesh(
    core_axis_name="c", subcore_axis_name="s", num_cores=NC, num_subcores=NS)


def _make_ring(rows, cols, table, acc_sh, ridx, cidx, cidx_loc, rowbuf,
               sem_i, sem_g, sem_s, cbase, base_c, hist_fn, NB, G):
    def issue_idx(b, k):
        off = (cbase + k) * K
        pltpu.async_copy(rows.at[pl.ds(off, K)], ridx.at[b], sem_i.at[b])
        pltpu.async_copy(cols.at[pl.ds(off, K)], cidx.at[b], sem_i.at[b])

    def wait_idx(b):
        pltpu.make_async_copy(rows.at[pl.ds(0, K)], ridx.at[b],
                              sem_i.at[b]).wait()
        pltpu.make_async_copy(cols.at[pl.ds(0, K)], cidx.at[b],
                              sem_i.at[b]).wait()

    def issue_gather(b):
        pltpu.async_copy(table.at[ridx.at[b]], rowbuf.at[b], sem_g.at[b])

    def wait_gather(b):
        pltpu.make_async_copy(table.at[ridx.at[b]], rowbuf.at[b],
                              sem_g.at[b]).wait()

    sidx = cidx if base_c is None else cidx_loc

    def issue_scatter(b):
        pltpu.async_copy(rowbuf.at[b], acc_sh.at[sidx.at[b]],
                         sem_s.at[b], add=True)

    def wait_scatter(b):
        pltpu.make_async_copy(rowbuf.at[b], acc_sh.at[sidx.at[b]],
                              sem_s.at[b]).wait()

    def localize(b):
        if base_c is None:
            return

        def grp(g, carry):
            v = cidx[b, pl.ds(g * 16, 16)]
            inr = (v >= base_c) & (v < base_c + HALF)
            cidx_loc[b, pl.ds(g * 16, 16)] = jnp.where(inr, v - base_c, DUMP)
            return carry
        lax.fori_loop(0, K // 16, grp, 0)

    def do_chunk(g, b, first, last):
        k = g * NB + b
        bn = (b + 1) % NB
        if (not first) or b == NB - 1:
            wait_scatter(bn)
        if not (last and b == NB - 1):
            wait_idx(bn)
            issue_gather(bn)
        localize(b)
        if hist_fn is not None:
            hist_fn(b)
        wait_gather(b)
        issue_scatter(b)
        if not last:
            issue_idx(b, k + NB)

    def run():
        for b in range(NB):
            issue_idx(b, b)
        wait_idx(0)
        issue_gather(0)
        for b in range(NB):
            do_chunk(0, b, True, False)

        def body(g, carry):
            for b in range(NB):
                do_chunk(g, b, False, False)
            return carry

        lax.fori_loop(1, G - 1, body, 0)
        for b in range(NB):
            do_chunk(G - 1, b, False, True)
        for b in range(1, NB):
            wait_scatter(b)

    return run


def _seg1_body(table, rows, cols, zbig, ssum_out, deg_out,
               acc_sh, hist_sh, ridx, cidx, cidx_loc, rowbuf, hist_v, tmp_v,
               red_v, degrows_v, sem_i, sem_g, sem_s):
    c = lax.axis_index("c")
    s = lax.axis_index("s")
    base_c = c * HALF
    b0 = s * RPT
    pltpu.sync_copy(zbig.at[pl.ds(b0, RPT)], acc_sh.at[pl.ds(b0, RPT)])

    def zhist(i, carry):
        hist_v[pl.ds(i * 16, 16)] = jnp.zeros((16,), jnp.float32)
        return carry

    lax.fori_loop(0, N_H // 16, zhist, 0)
    plsc.subcore_barrier()

    def hist_fn(b):
        def grp(g, carry2):
            v = cidx[b, pl.ds(g * 16, 16)]
            cnts, lastm = plsc.scan_count(v)
            plsc.addupdate_scatter(hist_v, [v], cnts.astype(jnp.float32),
                                   mask=lastm)
            return carry2

        lax.fori_loop(0, K // 16, grp, 0)

    _make_ring(rows, cols, table, acc_sh, ridx, cidx, cidx_loc, rowbuf,
               sem_i, sem_g, sem_s, s * CW1, base_c, hist_fn, NB1,
               CW1 // NB1)()

    pltpu.sync_copy(hist_v, hist_sh.at[pl.ds(s * N_H, N_H)])
    plsc.subcore_barrier()
    pltpu.sync_copy(acc_sh.at[pl.ds(b0, RPT)], ssum_out.at[c, pl.ds(b0, RPT)])

    w0 = base_c + b0
    pltpu.sync_copy(hist_sh.at[pl.ds(w0, RPT)], red_v)

    def comb(j, carry):
        pltpu.sync_copy(hist_sh.at[pl.ds(j * N_H + w0, RPT)], tmp_v)

        def addv(i, carry2):
            red_v[pl.ds(i * 16, 16)] = (red_v[pl.ds(i * 16, 16)]
                                        + tmp_v[pl.ds(i * 16, 16)])
            return carry2

        lax.fori_loop(0, RPT // 16, addv, 0)
        return carry

    lax.fori_loop(1, NS, comb, 0)

    for cblk in range(RPT // DB):
        def brow(r, carry):
            splat = plsc.load_gather(
                red_v, [jnp.full((16,), cblk * DB + r, jnp.int32)])

            def bcol(g, carry2):
                degrows_v[r, pl.ds(g * 16, 16)] = splat
                return carry2

            lax.fori_loop(0, H // 16, bcol, 0)
            return carry

        lax.fori_loop(0, DB, brow, 0)
        pltpu.sync_copy(degrows_v,
                        deg_out.at[c, pl.ds(b0 + cblk * DB, DB)])


_segsum_cnt = pl.kernel(
    _seg1_body,
    out_type=[jax.ShapeDtypeStruct((NC, ACC_ROWS, H), jnp.float32),
              jax.ShapeDtypeStruct((NC, ACC_ROWS, H), jnp.float32)],
    mesh=_mesh,
    compiler_params=pltpu.CompilerParams(needs_layout_passes=False),
    scratch_types=[
        pltpu.VMEM_SHARED((ACC_ROWS, H), jnp.float32),
        pltpu.VMEM_SHARED((NS * N_H,), jnp.float32),
        pltpu.VMEM((NB1, K), jnp.int32),
        pltpu.VMEM((NB1, K), jnp.int32),
        pltpu.VMEM((NB1, K), jnp.int32),
        pltpu.VMEM((NB1, K, H), jnp.float32),
        pltpu.VMEM((N_H,), jnp.float32),
        pltpu.VMEM((RPT,), jnp.float32),
        pltpu.VMEM((RPT,), jnp.float32),
        pltpu.VMEM((DB, H), jnp.float32),
        pltpu.SemaphoreType.DMA((NB1,)),
        pltpu.SemaphoreType.DMA((NB1,)),
        pltpu.SemaphoreType.DMA((NB1,)),
    ])


def _seg2_body(table, rows, cols, zbig2, ssum_out,
               acc_sh, ridx, cidx, rowbuf, sem_i, sem_g, sem_s):
    c = lax.axis_index("c")
    s = lax.axis_index("s")
    wid = s * NC + c
    b0 = s * RPT2
    pltpu.sync_copy(zbig2.at[pl.ds(b0, RPT2)], acc_sh.at[pl.ds(b0, RPT2)])
    plsc.subcore_barrier()

    _make_ring(rows, cols, table, acc_sh, ridx, cidx, None, rowbuf,
               sem_i, sem_g, sem_s, wid * CW2, None, None, NB2,
               CW2 // NB2)()

    plsc.subcore_barrier()
    pltpu.sync_copy(acc_sh.at[pl.ds(b0, RPT2)],
                    ssum_out.at[c, pl.ds(b0, RPT2)])


_segsum = pl.kernel(
    _seg2_body,
    out_type=[jax.ShapeDtypeStruct((NC, N_ACC2, H), jnp.float32)],
    mesh=_mesh,
    compiler_params=pltpu.CompilerParams(needs_layout_passes=False),
    scratch_types=[
        pltpu.VMEM_SHARED((N_ACC2, H), jnp.float32),
        pltpu.VMEM((NB2, K), jnp.int32),
        pltpu.VMEM((NB2, K), jnp.int32),
        pltpu.VMEM((NB2, K, H), jnp.float32),
        pltpu.SemaphoreType.DMA((NB2,)),
        pltpu.SemaphoreType.DMA((NB2,)),
        pltpu.SemaphoreType.DMA((NB2,)),
    ])


BN_ROWS = 1000
_GRID = N // BN_ROWS
_PB = HALF // BN_ROWS


def _stage_a(p_ref, d_ref, x_ref, Wl_ref, bl_ref, Wr_ref, We_ref, be_ref,
             Wg_ref, bg_ref, out_ref, st_ref, *, partials):
    i = pl.program_id(0)
    if partials:
        ssum = p_ref[0] + p_ref[1] + x_ref[...]
    else:
        ssum = p_ref[0] + x_ref[...]
    deg = d_ref[0] + 1.0
    aggr = ssum / deg
    out = (jnp.dot(aggr, Wl_ref[...], preferred_element_type=jnp.float32)
           + jnp.dot(x_ref[...], Wr_ref[...], preferred_element_type=jnp.float32)
           + bl_ref[...])
    ea = jnp.sum(We_ref[...], axis=0, keepdims=True) + be_ref[...]
    Wg = Wg_ref[...]
    cvec = jnp.dot(ea, Wg[H:], preferred_element_type=jnp.float32) + bg_ref[...]
    gate = jax.nn.sigmoid(
        jnp.dot(out, Wg[:H], preferred_element_type=jnp.float32) + cvec)
    out = out + gate * ea
    out_ref[...] = out
    s1 = jnp.sum(out, axis=0, keepdims=True)
    s2 = jnp.sum(out * out, axis=0, keepdims=True)
    st = jnp.concatenate([s1, s2, jnp.zeros((6, H), jnp.float32)], axis=0)

    @pl.when(i == 0)
    def _():
        st_ref[...] = st

    @pl.when(i > 0)
    def _():
        st_ref[...] = st_ref[...] + st


def _stage_b(out_ref, st_ref, bng_ref, bnb_ref, h_ref, st2_ref):
    i = pl.program_id(0)
    inv_n = 1.0 / N
    mu = st_ref[0:1] * inv_n
    var = st_ref[1:2] * inv_n - mu * mu
    rstd = lax.rsqrt(var + 1e-5)
    o = (out_ref[...] - mu) * (rstd * bng_ref[...]) + bnb_ref[...]
    h = jnp.maximum(o + o, 0.0)
    h_ref[...] = h
    s1 = jnp.sum(h, axis=0, keepdims=True)
    s2 = jnp.sum(h * h, axis=0, keepdims=True)
    st = jnp.concatenate([s1, s2, jnp.zeros((6, H), jnp.float32)], axis=0)

    @pl.when(i == 0)
    def _():
        st2_ref[...] = st

    @pl.when(i > 0)
    def _():
        st2_ref[...] = st2_ref[...] + st


def _gn_block(h, st2_ref, g_ref, b_ref, a_ref):
    inv_n = 1.0 / N
    mu = st2_ref[0:1] * inv_n
    ex2 = st2_ref[1:2] * inv_n
    a = a_ref[...]
    var = ex2 - (2.0 * a - a * a) * mu * mu
    xc = h - a * mu
    return jnp.maximum(g_ref[...] * xc * lax.rsqrt(var + 1e-5) + b_ref[...], 0.0)


def _stage_c(h_ref, st2_ref, g_ref, b_ref, a_ref, o_ref):
    o_ref[...] = _gn_block(h_ref[...], st2_ref, g_ref, b_ref, a_ref)


def _stage_c_final(h_ref, st2_ref, g_ref, b_ref, a_ref, W_ref, lb_ref, o_ref):
    hgn = _gn_block(h_ref[...], st2_ref, g_ref, b_ref, a_ref)
    o_ref[...] = jnp.dot(hgn, W_ref[...],
                         preferred_element_type=jnp.float32) + lb_ref[...]


def _row_spec():
    return pl.BlockSpec((BN_ROWS, H), lambda i: (i, 0))


def _half_spec():
    return pl.BlockSpec((1, BN_ROWS, H), lambda i: (i // _PB, i % _PB, 0))


def _full_spec(shape):
    nd = len(shape)
    return pl.BlockSpec(shape, lambda i: (0,) * nd)


def _dense_layer(ssum_p, deg_p, x, Wl, bl, Wr, We, be, Wg, bg, bng, bnb,
                 g, b, a, lin=None, partials=False):
    if partials:
        p_spec = pl.BlockSpec((NC, BN_ROWS, H), lambda i: (0, i, 0))
    else:
        p_spec = _half_spec()
    fa = pl.pallas_call(
        functools.partial(_stage_a, partials=partials),
        grid=(_GRID,),
        in_specs=[
            p_spec, _half_spec(), _row_spec(),
            _full_spec((H, H)), _full_spec((1, H)), _full_spec((H, H)),
            _full_spec((16, H)), _full_spec((1, H)),
            _full_spec((2 * H, H)), _full_spec((1, H)),
        ],
        out_specs=[_row_spec(), _full_spec((8, H))],
        out_shape=[jax.ShapeDtypeStruct((N, H), jnp.float32),
                   jax.ShapeDtypeStruct((8, H), jnp.float32)],
    )
    out, st = fa(ssum_p, deg_p, x, Wl, bl[None], Wr, We, be[None], Wg, bg[None])

    fb = pl.pallas_call(
        _stage_b,
        grid=(_GRID,),
        in_specs=[_row_spec(), _full_spec((8, H)),
                  _full_spec((1, H)), _full_spec((1, H))],
        out_specs=[_row_spec(), _full_spec((8, H))],
        out_shape=[jax.ShapeDtypeStruct((N, H), jnp.float32),
                   jax.ShapeDtypeStruct((8, H), jnp.float32)],
    )
    h, st2 = fb(out, st, bng[None], bnb[None])

    if lin is None:
        fc = pl.pallas_call(
            _stage_c,
            grid=(_GRID,),
            in_specs=[_row_spec(), _full_spec((8, H)),
                      _full_spec((1, H)), _full_spec((1, H)), _full_spec((1, H))],
            out_specs=_row_spec(),
            out_shape=jax.ShapeDtypeStruct((N, H), jnp.float32),
        )
        return fc(h, st2, g[None], b[None], a[None])
    lin_W, lin_b = lin
    fc = pl.pallas_call(
        _stage_c_final,
        grid=(_GRID,),
        in_specs=[_row_spec(), _full_spec((8, H)),
                  _full_spec((1, H)), _full_spec((1, H)), _full_spec((1, H)),
                  _full_spec((H, H)), _full_spec((1, H))],
        out_specs=_row_spec(),
        out_shape=jax.ShapeDtypeStruct((N, H), jnp.float32),
    )
    return fc(h, st2, g[None], b[None], a[None], lin_W, lin_b[None])


def kernel(x, edge_index, edge_attr, c1_Wl, c1_bl, c1_Wr, c1_We, c1_be, c1_Wg, c1_bg, c1_bng, c1_bnb, c2_Wl, c2_bl, c2_Wr, c2_We, c2_be, c2_Wg, c2_bg, c2_bng, c2_bnb, gn1_g, gn1_b, gn1_a, gn2_g, gn2_b, gn2_a, lin_W, lin_b):
    row = edge_index[0].astype(jnp.int32)
    col = edge_index[1].astype(jnp.int32)
    rows1 = jnp.concatenate([row, jnp.zeros((E_PAD1 - E,), jnp.int32)])
    cols1 = jnp.concatenate([col, jnp.full((E_PAD1 - E,), N, jnp.int32)])
    rows2 = jnp.concatenate([row, jnp.zeros((E_PAD2 - E,), jnp.int32)])
    cols2 = jnp.concatenate([col, jnp.full((E_PAD2 - E,), N, jnp.int32)])
    zbig = jnp.zeros((ACC_ROWS, H), jnp.float32)
    zbig2 = jnp.zeros((N_ACC2, H), jnp.float32)

    ssum1, deg1 = _segsum_cnt(x, rows1, cols1, zbig)
    h1 = _dense_layer(ssum1, deg1, x, c1_Wl, c1_bl, c1_Wr, c1_We, c1_be,
                      c1_Wg, c1_bg, c1_bng, c1_bnb, gn1_g, gn1_b, gn1_a)
    ssum2 = _segsum(h1, rows2, cols2, zbig2)
    if isinstance(ssum2, (list, tuple)):
        ssum2 = ssum2[0]
    return _dense_layer(ssum2, deg1, h1, c2_Wl, c2_bl, c2_Wr, c2_We, c2_be,
                        c2_Wg, c2_bg, c2_bng, c2_bnb, gn2_g, gn2_b, gn2_a,
                        lin=(lin_W, lin_b), partials=True)


# speedup vs baseline: 1.1018x; 1.1018x over previous
import functools

import jax
import jax.numpy as jnp
from jax import lax
from jax.experimental import pallas as pl
from jax.experimental.pallas import tpu as pltpu
from jax.experimental.pallas import tpu_sc as plsc

N = 10000
E = 320000
H = 128
NC = 2
NS = 16
K = 128
NB1 = 3
CW1 = 159
E_PAD1 = CW1 * NS * K
NB2 = 2
CW2 = 80
E_PAD2 = CW2 * NS * NC * K
HALF = N // NC
ACC_ROWS = 5120
DUMP = HALF + 8
RPT = ACC_ROWS // NS
DB = 16
N_H = 10128
N_ACC2 = 10112
RPT2 = N_ACC2 // NS

_mesh = plsc.VectorSubcoreMesh(
    core_axis_name="c", subcore_axis_name="s", num_cores=NC, num_subcores=NS)


def _make_ring(rows, cols, table, acc_sh, ridx, cidx, cidx_loc, rowbuf,
               sem_i, sem_g, sem_s, cbase, base_c, hist_fn, NB, G):
    def issue_idx(b, k):
        off = (cbase + k) * K
        pltpu.async_copy(rows.at[pl.ds(off, K)], ridx.at[b], sem_i.at[b])
        pltpu.async_copy(cols.at[pl.ds(off, K)], cidx.at[b], sem_i.at[b])

    def wait_idx(b):
        pltpu.make_async_copy(rows.at[pl.ds(0, K)], ridx.at[b],
                              sem_i.at[b]).wait()
        pltpu.make_async_copy(cols.at[pl.ds(0, K)], cidx.at[b],
                              sem_i.at[b]).wait()

    def issue_gather(b):
        pltpu.async_copy(table.at[ridx.at[b]], rowbuf.at[b], sem_g.at[b])

    def wait_gather(b):
        pltpu.make_async_copy(table.at[ridx.at[b]], rowbuf.at[b],
                              sem_g.at[b]).wait()

    sidx = cidx if base_c is None else cidx_loc

    def issue_scatter(b):
        pltpu.async_copy(rowbuf.at[b], acc_sh.at[sidx.at[b]],
                         sem_s.at[b], add=True)

    def wait_scatter(b):
        pltpu.make_async_copy(rowbuf.at[b], acc_sh.at[sidx.at[b]],
                              sem_s.at[b]).wait()

    def localize(b):
        if base_c is None:
            return

        def grp(g, carry):
            v = cidx[b, pl.ds(g * 16, 16)]
            inr = (v >= base_c) & (v < base_c + HALF)
            cidx_loc[b, pl.ds(g * 16, 16)] = jnp.where(inr, v - base_c, DUMP)
            return carry
        lax.fori_loop(0, K // 16, grp, 0)

    def do_chunk(g, b, first, last):
        k = g * NB + b
        bn = (b + 1) % NB
        if (not first) or b == NB - 1:
            wait_scatter(bn)
        if not (last and b == NB - 1):
            wait_idx(bn)
            issue_gather(bn)
        localize(b)
        if hist_fn is not None:
            hist_fn(b)
        wait_gather(b)
        issue_scatter(b)
        if not last:
            issue_idx(b, k + NB)

    def run():
        for b in range(NB):
            issue_idx(b, b)
        wait_idx(0)
        issue_gather(0)
        for b in range(NB):
            do_chunk(0, b, True, False)

        def body(g, carry):
            for b in range(NB):
                do_chunk(g, b, False, False)
            return carry

        lax.fori_loop(1, G - 1, body, 0)
        for b in range(NB):
            do_chunk(G - 1, b, False, True)
        for b in range(1, NB):
            wait_scatter(b)

    return run


BIGC = 1696


def _hist_body(cols, deg_out, hist_sh, cidx_big, hist_v, tmp_v, red_v,
               degrows_v):
    c = lax.axis_index("c")
    s = lax.axis_index("s")
    base_c = c * HALF
    b0 = s * RPT

    def zhist(i, carry):
        hist_v[pl.ds(i * 16, 16)] = jnp.zeros((16,), jnp.float32)
        return carry

    lax.fori_loop(0, N_H // 16, zhist, 0)

    def blk(k, carry):
        pltpu.sync_copy(cols.at[pl.ds(s * (CW1 * K) + k * BIGC, BIGC)],
                        cidx_big)

        def grp(g, carry2):
            v = cidx_big[pl.ds(g * 16, 16)]
            cnts, lastm = plsc.scan_count(v)
            plsc.addupdate_scatter(hist_v, [v], cnts.astype(jnp.float32),
                                   mask=lastm)
            return carry2

        lax.fori_loop(0, BIGC // 16, grp, 0)
        return carry

    lax.fori_loop(0, (CW1 * K) // BIGC, blk, 0)

    pltpu.sync_copy(hist_v, hist_sh.at[pl.ds(s * N_H, N_H)])
    plsc.subcore_barrier()

    w0 = base_c + b0
    pltpu.sync_copy(hist_sh.at[pl.ds(w0, RPT)], red_v)

    def comb(j, carry):
        pltpu.sync_copy(hist_sh.at[pl.ds(j * N_H + w0, RPT)], tmp_v)

        def addv(i, carry2):
            red_v[pl.ds(i * 16, 16)] = (red_v[pl.ds(i * 16, 16)]
                                        + tmp_v[pl.ds(i * 16, 16)])
            return carry2

        lax.fori_loop(0, RPT // 16, addv, 0)
        return carry

    lax.fori_loop(1, NS, comb, 0)

    for cblk in range(RPT // DB):
        def brow(r, carry):
            splat = plsc.load_gather(
                red_v, [jnp.full((16,), cblk * DB + r, jnp.int32)])

            def bcol(g, carry2):
                degrows_v[r, pl.ds(g * 16, 16)] = splat
                return carry2

            lax.fori_loop(0, H // 16, bcol, 0)
            return carry

        lax.fori_loop(0, DB, brow, 0)
        pltpu.sync_copy(degrows_v,
                        deg_out.at[c, pl.ds(b0 + cblk * DB, DB)])


_hist = pl.kernel(
    _hist_body,
    out_type=[jax.ShapeDtypeStruct((NC, ACC_ROWS, H), jnp.float32)],
    mesh=_mesh,
    compiler_params=pltpu.CompilerParams(needs_layout_passes=False),
    scratch_types=[
        pltpu.VMEM_SHARED((NS * N_H,), jnp.float32),
        pltpu.VMEM((BIGC,), jnp.int32),
        pltpu.VMEM((N_H,), jnp.float32),
        pltpu.VMEM((RPT,), jnp.float32),
        pltpu.VMEM((RPT,), jnp.float32),
        pltpu.VMEM((DB, H), jnp.float32),
    ])


def _seg2_body(table, rows, cols, zbig2, ssum_out,
               acc_sh, ridx, cidx, rowbuf, sem_i, sem_g, sem_s):
    c = lax.axis_index("c")
    s = lax.axis_index("s")
    wid = s * NC + c
    b0 = s * RPT2
    pltpu.sync_copy(zbig2.at[pl.ds(b0, RPT2)], acc_sh.at[pl.ds(b0, RPT2)])
    plsc.subcore_barrier()

    _make_ring(rows, cols, table, acc_sh, ridx, cidx, None, rowbuf,
               sem_i, sem_g, sem_s, wid * CW2, None, None, NB2,
               CW2 // NB2)()

    plsc.subcore_barrier()
    pltpu.sync_copy(acc_sh.at[pl.ds(b0, RPT2)],
                    ssum_out.at[c, pl.ds(b0, RPT2)])


_segsum = pl.kernel(
    _seg2_body,
    out_type=[jax.ShapeDtypeStruct((NC, N_ACC2, H), jnp.float32)],
    mesh=_mesh,
    compiler_params=pltpu.CompilerParams(needs_layout_passes=False),
    scratch_types=[
        pltpu.VMEM_SHARED((N_ACC2, H), jnp.float32),
        pltpu.VMEM((NB2, K), jnp.int32),
        pltpu.VMEM((NB2, K), jnp.int32),
        pltpu.VMEM((NB2, K, H), jnp.float32),
        pltpu.SemaphoreType.DMA((NB2,)),
        pltpu.SemaphoreType.DMA((NB2,)),
        pltpu.SemaphoreType.DMA((NB2,)),
    ])


BN_ROWS = 1000
_GRID = N // BN_ROWS
_PB = HALF // BN_ROWS


def _stage_a(p_ref, d_ref, x_ref, Wl_ref, bl_ref, Wr_ref, We_ref, be_ref,
             Wg_ref, bg_ref, out_ref, st_ref, *, partials):
    i = pl.program_id(0)
    if partials:
        ssum = p_ref[0] + p_ref[1] + x_ref[...]
    else:
        ssum = p_ref[0] + x_ref[...]
    deg = d_ref[0] + 1.0
    aggr = ssum / deg
    out = (jnp.dot(aggr, Wl_ref[...], preferred_element_type=jnp.float32)
           + jnp.dot(x_ref[...], Wr_ref[...], preferred_element_type=jnp.float32)
           + bl_ref[...])
    ea = jnp.sum(We_ref[...], axis=0, keepdims=True) + be_ref[...]
    Wg = Wg_ref[...]
    cvec = jnp.dot(ea, Wg[H:], preferred_element_type=jnp.float32) + bg_ref[...]
    gate = jax.nn.sigmoid(
        jnp.dot(out, Wg[:H], preferred_element_type=jnp.float32) + cvec)
    out = out + gate * ea
    out_ref[...] = out
    s1 = jnp.sum(out, axis=0, keepdims=True)
    s2 = jnp.sum(out * out, axis=0, keepdims=True)
    st = jnp.concatenate([s1, s2, jnp.zeros((6, H), jnp.float32)], axis=0)

    @pl.when(i == 0)
    def _():
        st_ref[...] = st

    @pl.when(i > 0)
    def _():
        st_ref[...] = st_ref[...] + st


def _stage_b(out_ref, st_ref, bng_ref, bnb_ref, h_ref, st2_ref):
    i = pl.program_id(0)
    inv_n = 1.0 / N
    mu = st_ref[0:1] * inv_n
    var = st_ref[1:2] * inv_n - mu * mu
    rstd = lax.rsqrt(var + 1e-5)
    o = (out_ref[...] - mu) * (rstd * bng_ref[...]) + bnb_ref[...]
    h = jnp.maximum(o + o, 0.0)
    h_ref[...] = h
    s1 = jnp.sum(h, axis=0, keepdims=True)
    s2 = jnp.sum(h * h, axis=0, keepdims=True)
    st = jnp.concatenate([s1, s2, jnp.zeros((6, H), jnp.float32)], axis=0)

    @pl.when(i == 0)
    def _():
        st2_ref[...] = st

    @pl.when(i > 0)
    def _():
        st2_ref[...] = st2_ref[...] + st


def _gn_block(h, st2_ref, g_ref, b_ref, a_ref):
    inv_n = 1.0 / N
    mu = st2_ref[0:1] * inv_n
    ex2 = st2_ref[1:2] * inv_n
    a = a_ref[...]
    var = ex2 - (2.0 * a - a * a) * mu * mu
    xc = h - a * mu
    return jnp.maximum(g_ref[...] * xc * lax.rsqrt(var + 1e-5) + b_ref[...], 0.0)


def _stage_c(h_ref, st2_ref, g_ref, b_ref, a_ref, o_ref):
    o_ref[...] = _gn_block(h_ref[...], st2_ref, g_ref, b_ref, a_ref)


def _stage_c_final(h_ref, st2_ref, g_ref, b_ref, a_ref, W_ref, lb_ref, o_ref):
    hgn = _gn_block(h_ref[...], st2_ref, g_ref, b_ref, a_ref)
    o_ref[...] = jnp.dot(hgn, W_ref[...],
                         preferred_element_type=jnp.float32) + lb_ref[...]


def _row_spec():
    return pl.BlockSpec((BN_ROWS, H), lambda i: (i, 0))


def _half_spec():
    return pl.BlockSpec((1, BN_ROWS, H), lambda i: (i // _PB, i % _PB, 0))


def _full_spec(shape):
    nd = len(shape)
    return pl.BlockSpec(shape, lambda i: (0,) * nd)


def _dense_layer(ssum_p, deg_p, x, Wl, bl, Wr, We, be, Wg, bg, bng, bnb,
                 g, b, a, lin=None, partials=False):
    if partials:
        p_spec = pl.BlockSpec((NC, BN_ROWS, H), lambda i: (0, i, 0))
    else:
        p_spec = _half_spec()
    fa = pl.pallas_call(
        functools.partial(_stage_a, partials=partials),
        grid=(_GRID,),
        in_specs=[
            p_spec, _half_spec(), _row_spec(),
            _full_spec((H, H)), _full_spec((1, H)), _full_spec((H, H)),
            _full_spec((16, H)), _full_spec((1, H)),
            _full_spec((2 * H, H)), _full_spec((1, H)),
        ],
        out_specs=[_row_spec(), _full_spec((8, H))],
        out_shape=[jax.ShapeDtypeStruct((N, H), jnp.float32),
                   jax.ShapeDtypeStruct((8, H), jnp.float32)],
    )
    out, st = fa(ssum_p, deg_p, x, Wl, bl[None], Wr, We, be[None], Wg, bg[None])

    fb = pl.pallas_call(
        _stage_b,
        grid=(_GRID,),
        in_specs=[_row_spec(), _full_spec((8, H)),
                  _full_spec((1, H)), _full_spec((1, H))],
        out_specs=[_row_spec(), _full_spec((8, H))],
        out_shape=[jax.ShapeDtypeStruct((N, H), jnp.float32),
                   jax.ShapeDtypeStruct((8, H), jnp.float32)],
    )
    h, st2 = fb(out, st, bng[None], bnb[None])

    if lin is None:
        fc = pl.pallas_call(
            _stage_c,
            grid=(_GRID,),
            in_specs=[_row_spec(), _full_spec((8, H)),
                      _full_spec((1, H)), _full_spec((1, H)), _full_spec((1, H))],
            out_specs=_row_spec(),
            out_shape=jax.ShapeDtypeStruct((N, H), jnp.float32),
        )
        return fc(h, st2, g[None], b[None], a[None])
    lin_W, lin_b = lin
    fc = pl.pallas_call(
        _stage_c_final,
        grid=(_GRID,),
        in_specs=[_row_spec(), _full_spec((8, H)),
                  _full_spec((1, H)), _full_spec((1, H)), _full_spec((1, H)),
                  _full_spec((H, H)), _full_spec((1, H))],
        out_specs=_row_spec(),
        out_shape=jax.ShapeDtypeStruct((N, H), jnp.float32),
    )
    return fc(h, st2, g[None], b[None], a[None], lin_W, lin_b[None])


def kernel(x, edge_index, edge_attr, c1_Wl, c1_bl, c1_Wr, c1_We, c1_be, c1_Wg, c1_bg, c1_bng, c1_bnb, c2_Wl, c2_bl, c2_Wr, c2_We, c2_be, c2_Wg, c2_bg, c2_bng, c2_bnb, gn1_g, gn1_b, gn1_a, gn2_g, gn2_b, gn2_a, lin_W, lin_b):
    row = edge_index[0].astype(jnp.int32)
    col = edge_index[1].astype(jnp.int32)
    cols1 = jnp.concatenate([col, jnp.full((E_PAD1 - E,), N, jnp.int32)])
    rows2 = jnp.concatenate([row, jnp.zeros((E_PAD2 - E,), jnp.int32)])
    cols2 = jnp.concatenate([col, jnp.full((E_PAD2 - E,), N, jnp.int32)])
    zbig2 = jnp.zeros((N_ACC2, H), jnp.float32)

    deg1 = _hist(cols1)
    if isinstance(deg1, (list, tuple)):
        deg1 = deg1[0]
    ssum1 = _segsum(x, rows2, cols2, zbig2)
    if isinstance(ssum1, (list, tuple)):
        ssum1 = ssum1[0]
    h1 = _dense_layer(ssum1, deg1, x, c1_Wl, c1_bl, c1_Wr, c1_We, c1_be,
                      c1_Wg, c1_bg, c1_bng, c1_bnb, gn1_g, gn1_b, gn1_a,
                      partials=True)
    ssum2 = _segsum(h1, rows2, cols2, zbig2)
    if isinstance(ssum2, (list, tuple)):
        ssum2 = ssum2[0]
    return _dense_layer(ssum2, deg1, h1, c2_Wl, c2_bl, c2_Wr, c2_We, c2_be,
                        c2_Wg, c2_bg, c2_bng, c2_bnb, gn2_g, gn2_b, gn2_a,
                        lin=(lin_W, lin_b), partials=True)


# speedup vs baseline: 1.1486x; 1.0424x over previous
import functools

import jax
import jax.numpy as jnp
from jax import lax
from jax.experimental import pallas as pl
from jax.experimental.pallas import tpu as pltpu
from jax.experimental.pallas import tpu_sc as plsc

N = 10000
E = 320000
H = 128
NC = 2
NS = 16
K = 128
NB1 = 3
CW1 = 159
E_PAD1 = CW1 * NS * K
NB2 = 2
CW2 = 80
E_PAD2 = CW2 * NS * NC * K
HALF = N // NC
ACC_ROWS = 5120
DUMP = HALF + 8
RPT = ACC_ROWS // NS
DB = 16
N_H = 10128
N_ACC2 = 10112
RPT2 = N_ACC2 // NS

_mesh = plsc.VectorSubcoreMesh(
    core_axis_name="c", subcore_axis_name="s", num_cores=NC, num_subcores=NS)


def _make_ring(rows, cols, table, acc_sh, ridx, cidx, cidx_loc, rowbuf,
               sem_i, sem_g, sem_s, cbase, base_c, hist_fn, NB, G):
    def issue_idx(b, k):
        off = (cbase + k) * K
        pltpu.async_copy(rows.at[pl.ds(off, K)], ridx.at[b], sem_i.at[b])
        pltpu.async_copy(cols.at[pl.ds(off, K)], cidx.at[b], sem_i.at[b])

    def wait_idx(b):
        pltpu.make_async_copy(rows.at[pl.ds(0, K)], ridx.at[b],
                              sem_i.at[b]).wait()
        pltpu.make_async_copy(cols.at[pl.ds(0, K)], cidx.at[b],
                              sem_i.at[b]).wait()

    def issue_gather(b):
        pltpu.async_copy(table.at[ridx.at[b]], rowbuf.at[b], sem_g.at[b])

    def wait_gather(b):
        pltpu.make_async_copy(table.at[ridx.at[b]], rowbuf.at[b],
                              sem_g.at[b]).wait()

    sidx = cidx if base_c is None else cidx_loc

    def issue_scatter(b):
        pltpu.async_copy(rowbuf.at[b], acc_sh.at[sidx.at[b]],
                         sem_s.at[b], add=True)

    def wait_scatter(b):
        pltpu.make_async_copy(rowbuf.at[b], acc_sh.at[sidx.at[b]],
                              sem_s.at[b]).wait()

    def localize(b):
        if base_c is None:
            return

        def grp(g, carry):
            v = cidx[b, pl.ds(g * 16, 16)]
            inr = (v >= base_c) & (v < base_c + HALF)
            cidx_loc[b, pl.ds(g * 16, 16)] = jnp.where(inr, v - base_c, DUMP)
            return carry
        lax.fori_loop(0, K // 16, grp, 0)

    def do_chunk(g, b, first, last):
        k = g * NB + b
        bn = (b + 1) % NB
        if (not first) or b == NB - 1:
            wait_scatter(bn)
        if not (last and b == NB - 1):
            wait_idx(bn)
            issue_gather(bn)
        localize(b)
        if hist_fn is not None:
            hist_fn(b)
        wait_gather(b)
        issue_scatter(b)
        if not last:
            issue_idx(b, k + NB)

    def run():
        for b in range(NB):
            issue_idx(b, b)
        wait_idx(0)
        issue_gather(0)
        for b in range(NB):
            do_chunk(0, b, True, False)

        def body(g, carry):
            for b in range(NB):
                do_chunk(g, b, False, False)
            return carry

        lax.fori_loop(1, G - 1, body, 0)
        for b in range(NB):
            do_chunk(G - 1, b, False, True)
        for b in range(1, NB):
            wait_scatter(b)

    return run


BIGC = 1696


def _hist_body(cols, deg_out, hist_sh, cidx_big, hist_v, tmp_v, red_v,
               degrows_v):
    c = lax.axis_index("c")
    s = lax.axis_index("s")
    base_c = c * HALF
    b0 = s * RPT

    def zhist(i, carry):
        hist_v[pl.ds(i * 16, 16)] = jnp.zeros((16,), jnp.float32)
        return carry

    lax.fori_loop(0, N_H // 16, zhist, 0)

    def blk(k, carry):
        pltpu.sync_copy(cols.at[pl.ds(s * (CW1 * K) + k * BIGC, BIGC)],
                        cidx_big)

        def grp(g, carry2):
            v = cidx_big[pl.ds(g * 16, 16)]
            cnts, lastm = plsc.scan_count(v)
            plsc.addupdate_scatter(hist_v, [v], cnts.astype(jnp.float32),
                                   mask=lastm)
            return carry2

        lax.fori_loop(0, BIGC // 16, grp, 0)
        return carry

    lax.fori_loop(0, (CW1 * K) // BIGC, blk, 0)

    pltpu.sync_copy(hist_v, hist_sh.at[pl.ds(s * N_H, N_H)])
    plsc.subcore_barrier()

    w0 = base_c + b0
    pltpu.sync_copy(hist_sh.at[pl.ds(w0, RPT)], red_v)

    def comb(j, carry):
        pltpu.sync_copy(hist_sh.at[pl.ds(j * N_H + w0, RPT)], tmp_v)

        def addv(i, carry2):
            red_v[pl.ds(i * 16, 16)] = (red_v[pl.ds(i * 16, 16)]
                                        + tmp_v[pl.ds(i * 16, 16)])
            return carry2

        lax.fori_loop(0, RPT // 16, addv, 0)
        return carry

    lax.fori_loop(1, NS, comb, 0)

    for cblk in range(RPT // DB):
        def brow(r, carry):
            splat = plsc.load_gather(
                red_v, [jnp.full((16,), cblk * DB + r, jnp.int32)])

            def bcol(g, carry2):
                degrows_v[r, pl.ds(g * 16, 16)] = splat
                return carry2

            lax.fori_loop(0, H // 16, bcol, 0)
            return carry

        lax.fori_loop(0, DB, brow, 0)
        pltpu.sync_copy(degrows_v,
                        deg_out.at[c, pl.ds(b0 + cblk * DB, DB)])


_hist = pl.kernel(
    _hist_body,
    out_type=[jax.ShapeDtypeStruct((NC, ACC_ROWS, H), jnp.float32)],
    mesh=_mesh,
    compiler_params=pltpu.CompilerParams(needs_layout_passes=False),
    scratch_types=[
        pltpu.VMEM_SHARED((NS * N_H,), jnp.float32),
        pltpu.VMEM((BIGC,), jnp.int32),
        pltpu.VMEM((N_H,), jnp.float32),
        pltpu.VMEM((RPT,), jnp.float32),
        pltpu.VMEM((RPT,), jnp.float32),
        pltpu.VMEM((DB, H), jnp.float32),
    ])


def _seg2_body(table, rows, cols, zbig2, ssum_out,
               acc_sh, ridx, cidx, rowbuf, sem_i, sem_g, sem_s):
    c = lax.axis_index("c")
    s = lax.axis_index("s")
    wid = s * NC + c
    b0 = s * RPT2
    pltpu.sync_copy(zbig2.at[pl.ds(b0, RPT2)], acc_sh.at[pl.ds(b0, RPT2)])
    plsc.subcore_barrier()

    _make_ring(rows, cols, table, acc_sh, ridx, cidx, None, rowbuf,
               sem_i, sem_g, sem_s, wid * CW2, None, None, NB2,
               CW2 // NB2)()

    plsc.subcore_barrier()
    pltpu.sync_copy(acc_sh.at[pl.ds(b0, RPT2)],
                    ssum_out.at[c, pl.ds(b0, RPT2)])


_segsum = pl.kernel(
    _seg2_body,
    out_type=[jax.ShapeDtypeStruct((NC, N_ACC2, H), jnp.float32)],
    mesh=_mesh,
    compiler_params=pltpu.CompilerParams(needs_layout_passes=False),
    scratch_types=[
        pltpu.VMEM_SHARED((N_ACC2, H), jnp.float32),
        pltpu.VMEM((NB2, K), jnp.int32),
        pltpu.VMEM((NB2, K), jnp.int32),
        pltpu.VMEM((NB2, K, H), jnp.float32),
        pltpu.SemaphoreType.DMA((NB2,)),
        pltpu.SemaphoreType.DMA((NB2,)),
        pltpu.SemaphoreType.DMA((NB2,)),
    ])


BN_ROWS = 1000
_GRID = N // BN_ROWS
_PB = HALF // BN_ROWS


def _stage_a(p_ref, d_ref, x_ref, Wl_ref, bl_ref, Wr_ref, We_ref, be_ref,
             Wg_ref, bg_ref, out_ref, st_ref, *, partials):
    i = pl.program_id(0)
    if partials:
        ssum = p_ref[0] + p_ref[1] + x_ref[...]
    else:
        ssum = p_ref[0] + x_ref[...]
    deg = d_ref[0] + 1.0
    aggr = ssum / deg
    out = (jnp.dot(aggr, Wl_ref[...], preferred_element_type=jnp.float32)
           + jnp.dot(x_ref[...], Wr_ref[...], preferred_element_type=jnp.float32)
           + bl_ref[...])
    ea = jnp.sum(We_ref[...], axis=0, keepdims=True) + be_ref[...]
    Wg = Wg_ref[...]
    cvec = jnp.dot(ea, Wg[H:], preferred_element_type=jnp.float32) + bg_ref[...]
    gate = jax.nn.sigmoid(
        jnp.dot(out, Wg[:H], preferred_element_type=jnp.float32) + cvec)
    out = out + gate * ea
    out_ref[...] = out
    s1 = jnp.sum(out, axis=0, keepdims=True)
    s2 = jnp.sum(out * out, axis=0, keepdims=True)
    st = jnp.concatenate([s1, s2, jnp.zeros((6, H), jnp.float32)], axis=0)

    @pl.when(i == 0)
    def _():
        st_ref[...] = st

    @pl.when(i > 0)
    def _():
        st_ref[...] = st_ref[...] + st


def _stage_b(out_ref, st_ref, bng_ref, bnb_ref, h_ref, st2_ref):
    i = pl.program_id(0)
    inv_n = 1.0 / N
    mu = st_ref[0:1] * inv_n
    var = st_ref[1:2] * inv_n - mu * mu
    rstd = lax.rsqrt(var + 1e-5)
    o = (out_ref[...] - mu) * (rstd * bng_ref[...]) + bnb_ref[...]
    h = jnp.maximum(o + o, 0.0)
    h_ref[...] = h
    s1 = jnp.sum(h, axis=0, keepdims=True)
    s2 = jnp.sum(h * h, axis=0, keepdims=True)
    st = jnp.concatenate([s1, s2, jnp.zeros((6, H), jnp.float32)], axis=0)

    @pl.when(i == 0)
    def _():
        st2_ref[...] = st

    @pl.when(i > 0)
    def _():
        st2_ref[...] = st2_ref[...] + st


def _gn_block(h, st2_ref, g_ref, b_ref, a_ref):
    inv_n = 1.0 / N
    mu = st2_ref[0:1] * inv_n
    ex2 = st2_ref[1:2] * inv_n
    a = a_ref[...]
    var = ex2 - (2.0 * a - a * a) * mu * mu
    xc = h - a * mu
    return jnp.maximum(g_ref[...] * xc * lax.rsqrt(var + 1e-5) + b_ref[...], 0.0)


def _stage_c(h_ref, st2_ref, g_ref, b_ref, a_ref, o_ref):
    o_ref[...] = _gn_block(h_ref[...], st2_ref, g_ref, b_ref, a_ref)


def _stage_c_final(h_ref, st2_ref, g_ref, b_ref, a_ref, W_ref, lb_ref, o_ref):
    hgn = _gn_block(h_ref[...], st2_ref, g_ref, b_ref, a_ref)
    o_ref[...] = jnp.dot(hgn, W_ref[...],
                         preferred_element_type=jnp.float32) + lb_ref[...]


def _row_spec():
    return pl.BlockSpec((BN_ROWS, H), lambda i: (i, 0))


def _half_spec():
    return pl.BlockSpec((1, BN_ROWS, H), lambda i: (i // _PB, i % _PB, 0))


def _full_spec(shape):
    nd = len(shape)
    return pl.BlockSpec(shape, lambda i: (0,) * nd)


def _dense_layer(ssum_p, deg_p, x, Wl, bl, Wr, We, be, Wg, bg, bng, bnb,
                 g, b, a, lin=None, partials=False):
    if partials:
        p_spec = pl.BlockSpec((NC, BN_ROWS, H), lambda i: (0, i, 0))
    else:
        p_spec = _half_spec()
    fa = pl.pallas_call(
        functools.partial(_stage_a, partials=partials),
        grid=(_GRID,),
        in_specs=[
            p_spec, _half_spec(), _row_spec(),
            _full_spec((H, H)), _full_spec((1, H)), _full_spec((H, H)),
            _full_spec((16, H)), _full_spec((1, H)),
            _full_spec((2 * H, H)), _full_spec((1, H)),
        ],
        out_specs=[_row_spec(), _full_spec((8, H))],
        out_shape=[jax.ShapeDtypeStruct((N, H), jnp.float32),
                   jax.ShapeDtypeStruct((8, H), jnp.float32)],
    )
    out, st = fa(ssum_p, deg_p, x, Wl, bl[None], Wr, We, be[None], Wg, bg[None])

    fb = pl.pallas_call(
        _stage_b,
        grid=(_GRID,),
        in_specs=[_row_spec(), _full_spec((8, H)),
                  _full_spec((1, H)), _full_spec((1, H))],
        out_specs=[_row_spec(), _full_spec((8, H))],
        out_shape=[jax.ShapeDtypeStruct((N, H), jnp.float32),
                   jax.ShapeDtypeStruct((8, H), jnp.float32)],
    )
    h, st2 = fb(out, st, bng[None], bnb[None])

    if lin is None:
        fc = pl.pallas_call(
            _stage_c,
            grid=(_GRID,),
            in_specs=[_row_spec(), _full_spec((8, H)),
                      _full_spec((1, H)), _full_spec((1, H)), _full_spec((1, H))],
            out_specs=_row_spec(),
            out_shape=jax.ShapeDtypeStruct((N, H), jnp.float32),
        )
        return fc(h, st2, g[None], b[None], a[None])
    lin_W, lin_b = lin
    fc = pl.pallas_call(
        _stage_c_final,
        grid=(_GRID,),
        in_specs=[_row_spec(), _full_spec((8, H)),
                  _full_spec((1, H)), _full_spec((1, H)), _full_spec((1, H)),
                  _full_spec((H, H)), _full_spec((1, H))],
        out_specs=_row_spec(),
        out_shape=jax.ShapeDtypeStruct((N, H), jnp.float32),
    )
    return fc(h, st2, g[None], b[None], a[None], lin_W, lin_b[None])


def kernel(x, edge_index, edge_attr, c1_Wl, c1_bl, c1_Wr, c1_We, c1_be, c1_Wg, c1_bg, c1_bng, c1_bnb, c2_Wl, c2_bl, c2_Wr, c2_We, c2_be, c2_Wg, c2_bg, c2_bng, c2_bnb, gn1_g, gn1_b, gn1_a, gn2_g, gn2_b, gn2_a, lin_W, lin_b):
    row = edge_index[0].astype(jnp.int32)
    col = edge_index[1].astype(jnp.int32)
    pad1 = N + jnp.arange(E_PAD1 - E, dtype=jnp.int32) % 96
    pad2 = N + jnp.arange(E_PAD2 - E, dtype=jnp.int32) % 96
    cols1 = jnp.concatenate([col, pad1])
    rows2 = jnp.concatenate([row, jnp.zeros((E_PAD2 - E,), jnp.int32)])
    cols2 = jnp.concatenate([col, pad2])
    zbig2 = jnp.zeros((N_ACC2, H), jnp.float32)

    deg1 = _hist(cols1)
    if isinstance(deg1, (list, tuple)):
        deg1 = deg1[0]
    ssum1 = _segsum(x, rows2, cols2, zbig2)
    if isinstance(ssum1, (list, tuple)):
        ssum1 = ssum1[0]
    h1 = _dense_layer(ssum1, deg1, x, c1_Wl, c1_bl, c1_Wr, c1_We, c1_be,
                      c1_Wg, c1_bg, c1_bng, c1_bnb, gn1_g, gn1_b, gn1_a,
                      partials=True)
    ssum2 = _segsum(h1, rows2, cols2, zbig2)
    if isinstance(ssum2, (list, tuple)):
        ssum2 = ssum2[0]
    return _dense_layer(ssum2, deg1, h1, c2_Wl, c2_bl, c2_Wr, c2_We, c2_be,
                        c2_Wg, c2_bg, c2_bng, c2_bnb, gn2_g, gn2_b, gn2_a,
                        lin=(lin_W, lin_b), partials=True)
